# baseline (speedup 1.0000x reference)
import functools
import math

import jax
import jax.numpy as jnp
from jax import lax
from jax.experimental import pallas as pl
from jax.experimental.pallas import tpu as pltpu
from jax.experimental.pallas import tpu_sc as plsc

_ROUTE_BUDGET = 8
_VALUE_BUDGET = 128

_N = 64
_S = 8192
_D = 128
_NB = 8
_SB = 512


def _qproj_body(q_ref, wq_ref, out_ref):
    out_ref[...] = jax.lax.dot_general(
        q_ref[...].astype(jnp.bfloat16), wq_ref[...].astype(jnp.bfloat16),
        (((1,), (1,)), ((), ())), preferred_element_type=jnp.float32)


def _f32_key(l):
    b = jax.lax.bitcast_convert_type(l, jnp.int32)
    return b ^ ((b >> 31) & jnp.int32(0x7FFFFFFF))


def _descend_t(key, t, bits, k):
    kk = jnp.int32(k)
    for bit in bits:
        inc = jnp.int32(-(2 ** 31)) if bit == 31 else jnp.int32(1 << bit)
        cand = t + inc
        cnt = jnp.sum((key >= cand).astype(jnp.int32), axis=-1, keepdims=True)
        t = jnp.where(cnt >= kk, cand, t)
    return t


def _descend_c(key, t, k):
    c_gt = jnp.sum((key > t).astype(jnp.int32), axis=-1, keepdims=True)
    m_eq = jnp.int32(k) - c_gt
    eq = key == t
    idx = jax.lax.broadcasted_iota(jnp.int32, key.shape, 1)
    c = jnp.zeros((key.shape[0], 1), dtype=jnp.int32)
    for bit in range(12, -1, -1):
        trial = c + jnp.int32((1 << bit) - 1)
        cnt = jnp.sum((eq & (idx <= trial)).astype(jnp.int32), axis=-1,
                      keepdims=True)
        c = jnp.where(cnt >= m_eq, c, c + jnp.int32(1 << bit))
    return c


def _weights_dest(l, key, t, c):
    rows = l.shape[0]
    idx = jax.lax.broadcasted_iota(jnp.int32, l.shape, 1)
    keep = (key > t) | ((key == t) & (idx <= c))
    mx = jnp.max(l, axis=-1, keepdims=True)
    e = jnp.where(keep, jnp.exp(l - mx), 0.0)
    z = jnp.sum(e, axis=-1, keepdims=True)
    w = e / z
    cum = keep.astype(jnp.int32)
    sh = 1
    while sh < l.shape[-1]:
        cum = cum + jnp.concatenate(
            [jnp.zeros((rows, sh), jnp.int32), cum[:, :-sh]], axis=-1)
        sh *= 2
    dest = jnp.where(keep, cum - 1, -1)
    return w, dest


_NBLK = _N // _NB
_SBLK = _S // _SB


def _fused_body(qp_ref, k_ref, v_ref, comb_ref, rw_ref, vw_ref, rd_ref,
                vd_ref, lr_ref, lv_ref, st_ref, *, scale):
    i = pl.program_id(0)
    j = pl.program_id(1)
    sl = lax.rem(i, 2)
    pv = 1 - sl

    @pl.when(i < _NBLK)
    def _logits():
        qp = qp_ref[...].astype(jnp.bfloat16)
        sel = jax.lax.broadcasted_iota(jnp.int32, (_NB, 1, _NB), 0) == \
            jax.lax.broadcasted_iota(jnp.int32, (_NB, 1, _NB), 2)

        def block_logits(x_ref):
            x = x_ref[...].reshape(_NB * _SB, _D).astype(jnp.bfloat16)
            allp = jax.lax.dot_general(x, qp, (((1,), (1,)), ((), ())),
                                       preferred_element_type=jnp.float32)
            allp = allp.reshape(_NB, _SB, _NB)
            return jnp.sum(jnp.where(sel, allp, 0.0), axis=-1) / scale

        lr_ref[sl, :, pl.ds(j * _SB, _SB)] = block_logits(k_ref)
        lv_ref[sl, :, pl.ds(j * _SB, _SB)] = block_logits(v_ref)

    @pl.when((i >= 1) & (j == 1))
    def _lump0():
        t0 = jnp.full((_NB, 1), -(2 ** 31), dtype=jnp.int32)
        st_ref[:, 0:1] = _descend_t(_f32_key(lr_ref[pv]), t0,
                                    range(31, 15, -1), _ROUTE_BUDGET)
        st_ref[:, 1:2] = _descend_t(_f32_key(lv_ref[pv]), t0,
                                    range(31, 15, -1), _VALUE_BUDGET)

    @pl.when((i >= 1) & (j == 5))
    def _lump1():
        st_ref[:, 0:1] = _descend_t(_f32_key(lr_ref[pv]), st_ref[:, 0:1],
                                    range(15, -1, -1), _ROUTE_BUDGET)
        st_ref[:, 1:2] = _descend_t(_f32_key(lv_ref[pv]), st_ref[:, 1:2],
                                    range(15, -1, -1), _VALUE_BUDGET)

    @pl.when((i >= 1) & (j == 9))
    def _lump2():
        st_ref[:, 2:3] = _descend_c(_f32_key(lr_ref[pv]), st_ref[:, 0:1],
                                    _ROUTE_BUDGET)
        st_ref[:, 3:4] = _descend_c(_f32_key(lv_ref[pv]), st_ref[:, 1:2],
                                    _VALUE_BUDGET)

    @pl.when((i >= 1) & (j == 13))
    def _lump3():
        rl = lr_ref[pv]
        vl = lv_ref[pv]
        rw, rd = _weights_dest(rl, _f32_key(rl), st_ref[:, 0:1],
                               st_ref[:, 2:3])
        vw, vd = _weights_dest(vl, _f32_key(vl), st_ref[:, 1:2],
                               st_ref[:, 3:4])
        comb_ref[...] = 0.5 * (rw + vw)
        rw_ref[...] = rw
        vw_ref[...] = vw
        rd_ref[...] = rd
        vd_ref[...] = vd


def _sc_branch(n, w_hbm, d_hbm, vflat_hbm, out_hbm, k, wrow, drow, idxbuf,
               wbuf, rowsbuf, outbuf, sem):
    pltpu.sync_copy(w_hbm.at[n], wrow)
    pltpu.sync_copy(d_hbm.at[n], drow)
    lanes = lax.broadcasted_iota(jnp.int32, (16,), 0)

    @plsc.parallel_loop(0, _S // 16)
    def _comp(c):
        wv = wrow[pl.ds(c * 16, 16)]
        dv = drow[pl.ds(c * 16, 16)]
        mask = dv >= 0
        gidx = lanes + (n * _S + c * 16)
        plsc.store_scatter(idxbuf, [dv], gidx, mask=mask)
        plsc.store_scatter(wbuf, [dv], wv, mask=mask)

    pltpu.async_copy(vflat_hbm.at[idxbuf], rowsbuf, sem).wait()

    def acc_body(j, accs):
        wbc = plsc.load_gather(wbuf, [jnp.full((16,), j, jnp.int32)])
        return tuple(accs[t] + wbc * rowsbuf[j, pl.ds(t * 16, 16)]
                     for t in range(_D // 16))

    accs = lax.fori_loop(0, k, acc_body,
                         tuple(jnp.zeros((16,), jnp.float32)
                               for _ in range(_D // 16)))
    for t in range(_D // 16):
        outbuf[pl.ds(t * 16, 16)] = accs[t]
    pltpu.sync_copy(outbuf, out_hbm.at[n])


def _sc_gather_body(rw_hbm, vw_hbm, rd_hbm, vd_hbm, vflat_hbm, rs_hbm,
                    vs_hbm, wrow, drow, idxv, wbufv, idxr, wbufr, rowsv,
                    rowsr, outbuf, sem):
    wid = lax.axis_index("s") * 2 + lax.axis_index("c")
    for r in range(_N // 32):
        n = wid * (_N // 32) + r
        _sc_branch(n, vw_hbm, vd_hbm, vflat_hbm, vs_hbm, _VALUE_BUDGET,
                   wrow, drow, idxv, wbufv, rowsv, outbuf, sem)
        _sc_branch(n, rw_hbm, rd_hbm, vflat_hbm, rs_hbm, _ROUTE_BUDGET,
                   wrow, drow, idxr, wbufr, rowsr, outbuf, sem)


def _head_body(rs_ref, vs_ref, qp_ref, wc_ref, bc_ref, wr_ref, br_ref,
               cls_ref, rec_ref, *, scale):
    rs = rs_ref[...]
    vs = vs_ref[...]
    qp = qp_ref[...]
    gate_logit = jnp.sum((rs - vs) * qp, axis=-1, keepdims=True) / scale
    gate = 1.0 / (1.0 + jnp.exp(-gate_logit))
    summary = gate * rs + (1.0 - gate) * vs
    cls_ref[...] = jax.lax.dot_general(
        summary, wc_ref[...], (((1,), (1,)), ((), ())),
        preferred_element_type=jnp.float32) + bc_ref[...]
    rec_ref[...] = jax.lax.dot_general(
        summary, wr_ref[...], (((1,), (1,)), ((), ())),
        preferred_element_type=jnp.float32) + br_ref[...]


def kernel(q, K, V, z, y, Wq, Wc, bc, Wr, br):
    del z, y
    scale = math.sqrt(_D)
    f32 = jnp.float32

    q_proj = pl.pallas_call(
        _qproj_body,
        out_shape=jax.ShapeDtypeStruct((_N, _D), f32),
    )(q, Wq)

    def _in_nb(i, j):
        return (jnp.minimum(i, _NBLK - 1), 0)

    def _in_kv(i, j):
        return (jnp.minimum(i, _NBLK - 1), j, 0)

    def _out_nb(i, j):
        return (jnp.maximum(i - 1, 0), 0)

    out_nb = pl.BlockSpec((_NB, _S), _out_nb)
    comb, rw, vw, rd, vd = pl.pallas_call(
        functools.partial(_fused_body, scale=scale),
        grid=(_NBLK + 1, _SBLK),
        in_specs=[
            pl.BlockSpec((_NB, _D), _in_nb),
            pl.BlockSpec((_NB, _SB, _D), _in_kv),
            pl.BlockSpec((_NB, _SB, _D), _in_kv),
        ],
        out_specs=[out_nb, out_nb, out_nb, out_nb, out_nb],
        out_shape=[
            jax.ShapeDtypeStruct((_N, _S), f32),
            jax.ShapeDtypeStruct((_N, _S), f32),
            jax.ShapeDtypeStruct((_N, _S), f32),
            jax.ShapeDtypeStruct((_N, _S), jnp.int32),
            jax.ShapeDtypeStruct((_N, _S), jnp.int32),
        ],
        scratch_shapes=[
            pltpu.VMEM((2, _NB, _S), f32),
            pltpu.VMEM((2, _NB, _S), f32),
            pltpu.VMEM((_NB, 128), jnp.int32),
        ],
    )(q_proj, K, V)

    sc_summaries = pl.kernel(
        _sc_gather_body,
        out_type=[
            jax.ShapeDtypeStruct((_N, _D), f32),
            jax.ShapeDtypeStruct((_N, _D), f32),
        ],
        mesh=plsc.VectorSubcoreMesh(core_axis_name="c", subcore_axis_name="s"),
        compiler_params=pltpu.CompilerParams(needs_layout_passes=False),
        scratch_types=[
            pltpu.VMEM((_S,), f32),
            pltpu.VMEM((_S,), jnp.int32),
            pltpu.VMEM((_VALUE_BUDGET,), jnp.int32),
            pltpu.VMEM((_VALUE_BUDGET,), f32),
            pltpu.VMEM((_ROUTE_BUDGET,), jnp.int32),
            pltpu.VMEM((_ROUTE_BUDGET,), f32),
            pltpu.VMEM((_VALUE_BUDGET, _D), f32),
            pltpu.VMEM((_ROUTE_BUDGET, _D), f32),
            pltpu.VMEM((_D,), f32),
            pltpu.SemaphoreType.DMA,
        ],
    )
    rs, vs = sc_summaries(rw, vw, rd, vd, V.reshape(_N * _S, _D))

    cls_out, recon_out = pl.pallas_call(
        functools.partial(_head_body, scale=scale),
        out_shape=[
            jax.ShapeDtypeStruct((_N, Wc.shape[0]), f32),
            jax.ShapeDtypeStruct((_N, _D), f32),
        ],
    )(rs, vs, q_proj, Wc, bc.reshape(1, -1), Wr, br.reshape(1, -1))

    return (cls_out, recon_out, comb)


# speedup vs baseline: 1.4461x; 1.4461x over previous
import functools
import math

import jax
import jax.numpy as jnp
from jax import lax
from jax.experimental import pallas as pl
from jax.experimental.pallas import tpu as pltpu
from jax.experimental.pallas import tpu_sc as plsc

_ROUTE_BUDGET = 8
_VALUE_BUDGET = 128

_N = 64
_S = 8192
_D = 128
_NB = 8
_SB = 1024


def _qproj_body(q_ref, wq_ref, out_ref):
    out_ref[...] = jax.lax.dot_general(
        q_ref[...].astype(jnp.bfloat16), wq_ref[...].astype(jnp.bfloat16),
        (((1,), (1,)), ((), ())), preferred_element_type=jnp.float32)


def _logits_body(qp_ref, k_ref, v_ref, rl_ref, vl_ref, *, scale):
    qp = qp_ref[...].astype(jnp.bfloat16)
    sel = jax.lax.broadcasted_iota(jnp.int32, (_NB, 1, _NB), 0) == \
        jax.lax.broadcasted_iota(jnp.int32, (_NB, 1, _NB), 2)

    def block_logits(x_ref):
        x = x_ref[...].reshape(_NB * _SB, _D).astype(jnp.bfloat16)
        allp = jax.lax.dot_general(x, qp, (((1,), (1,)), ((), ())),
                                   preferred_element_type=jnp.float32)
        allp = allp.reshape(_NB, _SB, _NB)
        return jnp.sum(jnp.where(sel, allp, 0.0), axis=-1) / scale

    rl_ref[...] = block_logits(k_ref)
    vl_ref[...] = block_logits(v_ref)


def _topk_weights_block(l, k):
    rows = l.shape[0]
    b = jax.lax.bitcast_convert_type(l, jnp.int32)
    key = b ^ ((b >> 31) & jnp.int32(0x7FFFFFFF))
    kk = jnp.int32(k)

    t = jnp.full((rows, 1), -(2 ** 31), dtype=jnp.int32)
    for bit in range(31, -1, -1):
        inc = jnp.int32(-(2 ** 31)) if bit == 31 else jnp.int32(1 << bit)
        cand = t + inc
        cnt = jnp.sum((key >= cand).astype(jnp.int32), axis=-1, keepdims=True)
        t = jnp.where(cnt >= kk, cand, t)

    c_gt = jnp.sum((key > t).astype(jnp.int32), axis=-1, keepdims=True)
    m_eq = kk - c_gt
    eq = key == t
    idx = jax.lax.broadcasted_iota(jnp.int32, l.shape, 1)
    c = jnp.zeros((rows, 1), dtype=jnp.int32)
    for bit in range(12, -1, -1):
        trial = c + jnp.int32((1 << bit) - 1)
        cnt = jnp.sum((eq & (idx <= trial)).astype(jnp.int32), axis=-1,
                      keepdims=True)
        c = jnp.where(cnt >= m_eq, c, c + jnp.int32(1 << bit))

    keep = (key > t) | (eq & (idx <= c))
    mx = jnp.max(l, axis=-1, keepdims=True)
    e = jnp.where(keep, jnp.exp(l - mx), 0.0)
    z = jnp.sum(e, axis=-1, keepdims=True)
    w = e / z
    cum = keep.astype(jnp.int32)
    sh = 1
    while sh < l.shape[-1]:
        cum = cum + jnp.concatenate(
            [jnp.zeros((rows, sh), jnp.int32), cum[:, :-sh]], axis=-1)
        sh *= 2
    dest = jnp.where(keep, cum - 1, -1)
    return w, dest


def _select_body(rl_ref, vl_ref, comb_ref, rw_ref, vw_ref, rd_ref, vd_ref):
    rw, rd = _topk_weights_block(rl_ref[...], _ROUTE_BUDGET)
    vw, vd = _topk_weights_block(vl_ref[...], _VALUE_BUDGET)
    comb_ref[...] = 0.5 * (rw + vw)
    rw_ref[...] = rw
    vw_ref[...] = vw
    rd_ref[...] = rd
    vd_ref[...] = vd


def _sc_branch(n, w_hbm, d_hbm, vflat_hbm, out_hbm, k, wrow, drow, idxbuf,
               wbuf, rowsbuf, outbuf, sem):
    pltpu.sync_copy(w_hbm.at[n], wrow)
    pltpu.sync_copy(d_hbm.at[n], drow)
    lanes = lax.broadcasted_iota(jnp.int32, (16,), 0)

    @plsc.parallel_loop(0, _S // 16)
    def _comp(c):
        wv = wrow[pl.ds(c * 16, 16)]
        dv = drow[pl.ds(c * 16, 16)]
        mask = dv >= 0
        gidx = lanes + (n * _S + c * 16)
        plsc.store_scatter(idxbuf, [dv], gidx, mask=mask)
        plsc.store_scatter(wbuf, [dv], wv, mask=mask)

    pltpu.async_copy(vflat_hbm.at[idxbuf], rowsbuf, sem).wait()

    def acc_body(j, accs):
        wbc = plsc.load_gather(wbuf, [jnp.full((16,), j, jnp.int32)])
        return tuple(accs[t] + wbc * rowsbuf[j, pl.ds(t * 16, 16)]
                     for t in range(_D // 16))

    accs = lax.fori_loop(0, k, acc_body,
                         tuple(jnp.zeros((16,), jnp.float32)
                               for _ in range(_D // 16)))
    for t in range(_D // 16):
        outbuf[pl.ds(t * 16, 16)] = accs[t]
    pltpu.sync_copy(outbuf, out_hbm.at[n])


def _sc_gather_body(rw_hbm, vw_hbm, rd_hbm, vd_hbm, vflat_hbm, rs_hbm,
                    vs_hbm, wrow, drow, idxv, wbufv, idxr, wbufr, rowsv,
                    rowsr, outbuf, sem):
    wid = lax.axis_index("s") * 2 + lax.axis_index("c")
    for r in range(_N // 32):
        n = wid * (_N // 32) + r
        _sc_branch(n, vw_hbm, vd_hbm, vflat_hbm, vs_hbm, _VALUE_BUDGET,
                   wrow, drow, idxv, wbufv, rowsv, outbuf, sem)
        _sc_branch(n, rw_hbm, rd_hbm, vflat_hbm, rs_hbm, _ROUTE_BUDGET,
                   wrow, drow, idxr, wbufr, rowsr, outbuf, sem)


def _head_body(rs_ref, vs_ref, qp_ref, wc_ref, bc_ref, wr_ref, br_ref,
               cls_ref, rec_ref, *, scale):
    rs = rs_ref[...]
    vs = vs_ref[...]
    qp = qp_ref[...]
    gate_logit = jnp.sum((rs - vs) * qp, axis=-1, keepdims=True) / scale
    gate = 1.0 / (1.0 + jnp.exp(-gate_logit))
    summary = gate * rs + (1.0 - gate) * vs
    cls_ref[...] = jax.lax.dot_general(
        summary, wc_ref[...], (((1,), (1,)), ((), ())),
        preferred_element_type=jnp.float32) + bc_ref[...]
    rec_ref[...] = jax.lax.dot_general(
        summary, wr_ref[...], (((1,), (1,)), ((), ())),
        preferred_element_type=jnp.float32) + br_ref[...]


def kernel(q, K, V, z, y, Wq, Wc, bc, Wr, br):
    del z, y
    scale = math.sqrt(_D)
    f32 = jnp.float32

    q_proj = pl.pallas_call(
        _qproj_body,
        out_shape=jax.ShapeDtypeStruct((_N, _D), f32),
    )(q, Wq)

    n_blocks = _N // _NB
    s_blocks = _S // _SB
    rl, vl = pl.pallas_call(
        functools.partial(_logits_body, scale=scale),
        grid=(n_blocks, s_blocks),
        in_specs=[
            pl.BlockSpec((_NB, _D), lambda i, j: (i, 0)),
            pl.BlockSpec((_NB, _SB, _D), lambda i, j: (i, j, 0)),
            pl.BlockSpec((_NB, _SB, _D), lambda i, j: (i, j, 0)),
        ],
        out_specs=[
            pl.BlockSpec((_NB, _SB), lambda i, j: (i, j)),
            pl.BlockSpec((_NB, _SB), lambda i, j: (i, j)),
        ],
        out_shape=[
            jax.ShapeDtypeStruct((_N, _S), f32),
            jax.ShapeDtypeStruct((_N, _S), f32),
        ],
    )(q_proj, K, V)

    comb, rw, vw, rd, vd = pl.pallas_call(
        _select_body,
        out_shape=[
            jax.ShapeDtypeStruct((_N, _S), f32),
            jax.ShapeDtypeStruct((_N, _S), f32),
            jax.ShapeDtypeStruct((_N, _S), f32),
            jax.ShapeDtypeStruct((_N, _S), jnp.int32),
            jax.ShapeDtypeStruct((_N, _S), jnp.int32),
        ],
    )(rl, vl)

    sc_summaries = pl.kernel(
        _sc_gather_body,
        out_type=[
            jax.ShapeDtypeStruct((_N, _D), f32),
            jax.ShapeDtypeStruct((_N, _D), f32),
        ],
        mesh=plsc.VectorSubcoreMesh(core_axis_name="c", subcore_axis_name="s"),
        compiler_params=pltpu.CompilerParams(needs_layout_passes=False),
        scratch_types=[
            pltpu.VMEM((_S,), f32),
            pltpu.VMEM((_S,), jnp.int32),
            pltpu.VMEM((_VALUE_BUDGET,), jnp.int32),
            pltpu.VMEM((_VALUE_BUDGET,), f32),
            pltpu.VMEM((_ROUTE_BUDGET,), jnp.int32),
            pltpu.VMEM((_ROUTE_BUDGET,), f32),
            pltpu.VMEM((_VALUE_BUDGET, _D), f32),
            pltpu.VMEM((_ROUTE_BUDGET, _D), f32),
            pltpu.VMEM((_D,), f32),
            pltpu.SemaphoreType.DMA,
        ],
    )
    rs, vs = sc_summaries(rw, vw, rd, vd, V.reshape(_N * _S, _D))

    cls_out, recon_out = pl.pallas_call(
        functools.partial(_head_body, scale=scale),
        out_shape=[
            jax.ShapeDtypeStruct((_N, Wc.shape[0]), f32),
            jax.ShapeDtypeStruct((_N, _D), f32),
        ],
    )(rs, vs, q_proj, Wc, bc.reshape(1, -1), Wr, br.reshape(1, -1))

    return (cls_out, recon_out, comb)


# speedup vs baseline: 1.5552x; 1.0754x over previous
import functools
import math

import jax
import jax.numpy as jnp
from jax import lax
from jax.experimental import pallas as pl
from jax.experimental.pallas import tpu as pltpu
from jax.experimental.pallas import tpu_sc as plsc

_ROUTE_BUDGET = 8
_VALUE_BUDGET = 128

_N = 64
_S = 8192
_D = 128
_NB = 8
_SB = 2048


def _qproj_body(q_ref, wq_ref, out_ref):
    out_ref[...] = jax.lax.dot_general(
        q_ref[...].astype(jnp.bfloat16), wq_ref[...].astype(jnp.bfloat16),
        (((1,), (1,)), ((), ())), preferred_element_type=jnp.float32)


def _logits_body(qp_ref, k_ref, v_ref, rl_ref, vl_ref, *, scale):
    qp = qp_ref[...].astype(jnp.bfloat16)
    sel = jax.lax.broadcasted_iota(jnp.int32, (_NB, 1, _NB), 0) == \
        jax.lax.broadcasted_iota(jnp.int32, (_NB, 1, _NB), 2)

    def block_logits(x_ref):
        x = x_ref[...].reshape(_NB * _SB, _D).astype(jnp.bfloat16)
        allp = jax.lax.dot_general(x, qp, (((1,), (1,)), ((), ())),
                                   preferred_element_type=jnp.float32)
        allp = allp.reshape(_NB, _SB, _NB)
        return jnp.sum(jnp.where(sel, allp, 0.0), axis=-1) / scale

    rl_ref[...] = block_logits(k_ref)
    vl_ref[...] = block_logits(v_ref)


def _topk_weights_block(l, k):
    rows = l.shape[0]
    b = jax.lax.bitcast_convert_type(l, jnp.int32)
    key = b ^ ((b >> 31) & jnp.int32(0x7FFFFFFF))
    kk = jnp.int32(k)

    t = jnp.full((rows, 1), -(2 ** 31), dtype=jnp.int32)
    for bit in range(31, -1, -1):
        inc = jnp.int32(-(2 ** 31)) if bit == 31 else jnp.int32(1 << bit)
        cand = t + inc
        cnt = jnp.sum((key >= cand).astype(jnp.int32), axis=-1, keepdims=True)
        t = jnp.where(cnt >= kk, cand, t)

    c_gt = jnp.sum((key > t).astype(jnp.int32), axis=-1, keepdims=True)
    m_eq = kk - c_gt
    eq = key == t
    idx = jax.lax.broadcasted_iota(jnp.int32, l.shape, 1)
    c = jnp.zeros((rows, 1), dtype=jnp.int32)
    for bit in range(12, -1, -1):
        trial = c + jnp.int32((1 << bit) - 1)
        cnt = jnp.sum((eq & (idx <= trial)).astype(jnp.int32), axis=-1,
                      keepdims=True)
        c = jnp.where(cnt >= m_eq, c, c + jnp.int32(1 << bit))

    keep = (key > t) | (eq & (idx <= c))
    mx = jnp.max(l, axis=-1, keepdims=True)
    e = jnp.where(keep, jnp.exp(l - mx), 0.0)
    z = jnp.sum(e, axis=-1, keepdims=True)
    w = e / z
    cum = keep.astype(jnp.int32)
    sh = 1
    while sh < l.shape[-1]:
        cum = cum + jnp.concatenate(
            [jnp.zeros((rows, sh), jnp.int32), cum[:, :-sh]], axis=-1)
        sh *= 2
    dest = jnp.where(keep, cum - 1, -1)
    return w, dest


def _select_body(rl_ref, vl_ref, comb_ref, rw_ref, vw_ref, rd_ref, vd_ref):
    rw, rd = _topk_weights_block(rl_ref[...], _ROUTE_BUDGET)
    vw, vd = _topk_weights_block(vl_ref[...], _VALUE_BUDGET)
    comb_ref[...] = 0.5 * (rw + vw)
    rw_ref[...] = rw
    vw_ref[...] = vw
    rd_ref[...] = rd
    vd_ref[...] = vd


def _sc_branch(n, w_hbm, d_hbm, vflat_hbm, out_hbm, k, wrow, drow, idxbuf,
               wbuf, rowsbuf, outbuf, sem):
    pltpu.sync_copy(w_hbm.at[n], wrow)
    pltpu.sync_copy(d_hbm.at[n], drow)
    lanes = lax.broadcasted_iota(jnp.int32, (16,), 0)

    @plsc.parallel_loop(0, _S // 16)
    def _comp(c):
        wv = wrow[pl.ds(c * 16, 16)]
        dv = drow[pl.ds(c * 16, 16)]
        mask = dv >= 0
        gidx = lanes + (n * _S + c * 16)
        plsc.store_scatter(idxbuf, [dv], gidx, mask=mask)
        plsc.store_scatter(wbuf, [dv], wv, mask=mask)

    pltpu.async_copy(vflat_hbm.at[idxbuf], rowsbuf, sem).wait()

    def acc_body(j, accs):
        wbc = plsc.load_gather(wbuf, [jnp.full((16,), j, jnp.int32)])
        return tuple(accs[t] + wbc * rowsbuf[j, pl.ds(t * 16, 16)]
                     for t in range(_D // 16))

    accs = lax.fori_loop(0, k, acc_body,
                         tuple(jnp.zeros((16,), jnp.float32)
                               for _ in range(_D // 16)))
    for t in range(_D // 16):
        outbuf[pl.ds(t * 16, 16)] = accs[t]
    pltpu.sync_copy(outbuf, out_hbm.at[n])


def _sc_gather_body(rw_hbm, vw_hbm, rd_hbm, vd_hbm, vflat_hbm, rs_hbm,
                    vs_hbm, wrow, drow, idxv, wbufv, idxr, wbufr, rowsv,
                    rowsr, outbuf, sem):
    wid = lax.axis_index("s") * 2 + lax.axis_index("c")
    for r in range(_N // 32):
        n = wid * (_N // 32) + r
        _sc_branch(n, vw_hbm, vd_hbm, vflat_hbm, vs_hbm, _VALUE_BUDGET,
                   wrow, drow, idxv, wbufv, rowsv, outbuf, sem)
        _sc_branch(n, rw_hbm, rd_hbm, vflat_hbm, rs_hbm, _ROUTE_BUDGET,
                   wrow, drow, idxr, wbufr, rowsr, outbuf, sem)


def _head_body(rs_ref, vs_ref, qp_ref, wc_ref, bc_ref, wr_ref, br_ref,
               cls_ref, rec_ref, *, scale):
    rs = rs_ref[...]
    vs = vs_ref[...]
    qp = qp_ref[...]
    gate_logit = jnp.sum((rs - vs) * qp, axis=-1, keepdims=True) / scale
    gate = 1.0 / (1.0 + jnp.exp(-gate_logit))
    summary = gate * rs + (1.0 - gate) * vs
    cls_ref[...] = jax.lax.dot_general(
        summary, wc_ref[...], (((1,), (1,)), ((), ())),
        preferred_element_type=jnp.float32) + bc_ref[...]
    rec_ref[...] = jax.lax.dot_general(
        summary, wr_ref[...], (((1,), (1,)), ((), ())),
        preferred_element_type=jnp.float32) + br_ref[...]


def kernel(q, K, V, z, y, Wq, Wc, bc, Wr, br):
    del z, y
    scale = math.sqrt(_D)
    f32 = jnp.float32

    q_proj = pl.pallas_call(
        _qproj_body,
        out_shape=jax.ShapeDtypeStruct((_N, _D), f32),
    )(q, Wq)

    n_blocks = _N // _NB
    s_blocks = _S // _SB
    rl, vl = pl.pallas_call(
        functools.partial(_logits_body, scale=scale),
        grid=(n_blocks, s_blocks),
        in_specs=[
            pl.BlockSpec((_NB, _D), lambda i, j: (i, 0)),
            pl.BlockSpec((_NB, _SB, _D), lambda i, j: (i, j, 0)),
            pl.BlockSpec((_NB, _SB, _D), lambda i, j: (i, j, 0)),
        ],
        out_specs=[
            pl.BlockSpec((_NB, _SB), lambda i, j: (i, j)),
            pl.BlockSpec((_NB, _SB), lambda i, j: (i, j)),
        ],
        out_shape=[
            jax.ShapeDtypeStruct((_N, _S), f32),
            jax.ShapeDtypeStruct((_N, _S), f32),
        ],
    )(q_proj, K, V)

    comb, rw, vw, rd, vd = pl.pallas_call(
        _select_body,
        out_shape=[
            jax.ShapeDtypeStruct((_N, _S), f32),
            jax.ShapeDtypeStruct((_N, _S), f32),
            jax.ShapeDtypeStruct((_N, _S), f32),
            jax.ShapeDtypeStruct((_N, _S), jnp.int32),
            jax.ShapeDtypeStruct((_N, _S), jnp.int32),
        ],
    )(rl, vl)

    sc_summaries = pl.kernel(
        _sc_gather_body,
        out_type=[
            jax.ShapeDtypeStruct((_N, _D), f32),
            jax.ShapeDtypeStruct((_N, _D), f32),
        ],
        mesh=plsc.VectorSubcoreMesh(core_axis_name="c", subcore_axis_name="s"),
        compiler_params=pltpu.CompilerParams(needs_layout_passes=False),
        scratch_types=[
            pltpu.VMEM((_S,), f32),
            pltpu.VMEM((_S,), jnp.int32),
            pltpu.VMEM((_VALUE_BUDGET,), jnp.int32),
            pltpu.VMEM((_VALUE_BUDGET,), f32),
            pltpu.VMEM((_ROUTE_BUDGET,), jnp.int32),
            pltpu.VMEM((_ROUTE_BUDGET,), f32),
            pltpu.VMEM((_VALUE_BUDGET, _D), f32),
            pltpu.VMEM((_ROUTE_BUDGET, _D), f32),
            pltpu.VMEM((_D,), f32),
            pltpu.SemaphoreType.DMA,
        ],
    )
    rs, vs = sc_summaries(rw, vw, rd, vd, V.reshape(_N * _S, _D))

    cls_out, recon_out = pl.pallas_call(
        functools.partial(_head_body, scale=scale),
        out_shape=[
            jax.ShapeDtypeStruct((_N, Wc.shape[0]), f32),
            jax.ShapeDtypeStruct((_N, _D), f32),
        ],
    )(rs, vs, q_proj, Wc, bc.reshape(1, -1), Wr, br.reshape(1, -1))

    return (cls_out, recon_out, comb)
